# baseline (speedup 1.0000x reference)
import functools

import jax
import jax.numpy as jnp
from jax import lax
from jax.experimental import pallas as pl
from jax.experimental.pallas import tpu as pltpu
from jax.experimental.pallas import tpu_sc as plsc

N = 10000
E = 160000
F = 256
H2 = 128
FH = 128
NC, NS = 2, 16
RPT = N // NS

EPW = E // (NC * NS)
KD = 40
NCHD = EPW // KD

EPT = E // NS
K = 80
NCHK = EPT // K

_MESH = plsc.VectorSubcoreMesh(core_axis_name="c", subcore_axis_name="s",
                               num_cores=NC, num_subcores=NS)


@functools.partial(
    pl.kernel,
    out_type=jax.ShapeDtypeStruct((NC * NS, RPT, 16), jnp.float32),
    mesh=_MESH,
    scratch_types=[
        pltpu.VMEM((NCHD, KD), jnp.int32),
        pltpu.VMEM((KD, 16), jnp.float32),
        pltpu.VMEM_SHARED((N, 16), jnp.float32),
    ],
    compiler_params=pltpu.CompilerParams(use_tc_tiling_on_sc=False),
)
def _sc_degree(dst_hbm, ones_hbm, zeros_hbm, out_hbm, dst_v, ones_v, acc):
    c = lax.axis_index("c")
    s = lax.axis_index("s")
    wid = c * NS + s
    pltpu.sync_copy(dst_hbm.at[wid], dst_v)
    pltpu.sync_copy(ones_hbm, ones_v)
    pltpu.sync_copy(zeros_hbm, acc.at[pl.ds(s * RPT, RPT)])
    plsc.subcore_barrier()

    def body(j, carry):
        pltpu.sync_copy(ones_v, acc.at[dst_v.at[j]], add=True)
        return carry

    lax.fori_loop(0, NCHD, body, 0)
    plsc.subcore_barrier()
    pltpu.sync_copy(acc.at[pl.ds(s * RPT, RPT)], out_hbm.at[wid])


@functools.partial(
    pl.kernel,
    out_type=jax.ShapeDtypeStruct((NC * NS, RPT, FH), jnp.float32),
    mesh=_MESH,
    scratch_types=[
        pltpu.VMEM((NCHK, K), jnp.int32),
        pltpu.VMEM((NCHK, K), jnp.int32),
        pltpu.VMEM((K, FH), jnp.float32),
        pltpu.VMEM((K, FH), jnp.float32),
        pltpu.VMEM_SHARED((N, FH), jnp.float32),
        pltpu.SemaphoreType.DMA,
        pltpu.SemaphoreType.DMA,
        pltpu.SemaphoreType.DMA,
        pltpu.SemaphoreType.DMA,
    ],
    compiler_params=pltpu.CompilerParams(use_tc_tiling_on_sc=False),
)
def _sc_scatter(y_hbm, src2_hbm, dst_hbm, zeros_hbm, out_hbm,
                src_v, dst_v, rows0, rows1, acc, g0, g1, s0, s1):
    c = lax.axis_index("c")
    s = lax.axis_index("s")
    wid = c * NS + s
    pltpu.sync_copy(src2_hbm.at[c, s], src_v)
    pltpu.sync_copy(dst_hbm.at[s], dst_v)
    pltpu.sync_copy(zeros_hbm, acc.at[pl.ds(s * RPT, RPT)])
    plsc.subcore_barrier()

    def gather(rows, sem, j):
        pltpu.async_copy(y_hbm.at[src_v.at[j]], rows, sem)

    def wait_gather(rows, sem, j):
        pltpu.make_async_copy(y_hbm.at[src_v.at[j]], rows, sem).wait()

    def scatter(rows, sem, j):
        pltpu.async_copy(rows, acc.at[dst_v.at[j]], sem, add=True)

    def wait_scatter(rows, sem, j):
        pltpu.make_async_copy(rows, acc.at[dst_v.at[j]], sem).wait()

    gather(rows0, g0, 0)
    gather(rows1, g1, 1)
    wait_gather(rows0, g0, 0)
    scatter(rows0, s0, 0)

    def body(t2, carry):
        t = 2 * t2
        wait_gather(rows1, g1, t + 1)
        scatter(rows1, s1, t + 1)
        wait_scatter(rows0, s0, t)
        gather(rows0, g0, t + 2)
        wait_gather(rows0, g0, t + 2)
        scatter(rows0, s0, t + 2)
        wait_scatter(rows1, s1, t + 1)
        gather(rows1, g1, t + 3)
        return carry

    lax.fori_loop(0, (NCHK - 3) // 2, body, 0)
    t = NCHK - 3
    wait_gather(rows1, g1, t + 1)
    scatter(rows1, s1, t + 1)
    wait_scatter(rows0, s0, t)
    gather(rows0, g0, t + 2)
    wait_gather(rows0, g0, t + 2)
    scatter(rows0, s0, t + 2)
    wait_scatter(rows1, s1, t + 1)
    wait_scatter(rows0, s0, t + 2)
    plsc.subcore_barrier()
    pltpu.sync_copy(acc.at[pl.ds(s * RPT, RPT)], out_hbm.at[wid])


_BR = 1000


def _dinv(dp):
    return lax.rsqrt(dp[0, :, :1] + dp[1, :, :1] + 1.0)


def _tc_xw_body(x_ref, w_ref, dp_ref, y_ref):
    dinv = _dinv(dp_ref[...])
    y_ref[...] = jnp.dot(x_ref[...], w_ref[...],
                         preferred_element_type=jnp.float32) * dinv


def _tc_layer2_body(s1_ref, y1_ref, dp_ref, b1_ref, w_ref, y2_ref):
    dinv = _dinv(dp_ref[...])
    s1 = jnp.concatenate([s1_ref[0], s1_ref[1]], axis=-1)
    h = jnp.maximum(dinv * (s1 + y1_ref[...]) + b1_ref[...], 0.0)
    y2_ref[...] = jnp.dot(h, w_ref[...],
                          preferred_element_type=jnp.float32) * dinv


def _tc_out_body(s2_ref, y2_ref, dp_ref, b_ref, o_ref):
    dinv = _dinv(dp_ref[...])
    s2 = jnp.concatenate([s2_ref[0], s2_ref[1]], axis=-1)
    o_ref[...] = dinv * (s2 + y2_ref[...]) + b_ref[...]


_BD_I = 2000
_BD_J = 2048


def _tc_dec_body(a_ref, b_ref, o_ref):
    logits = lax.dot_general(a_ref[...], b_ref[...],
                             (((1,), (1,)), ((), ())),
                             preferred_element_type=jnp.float32)
    o_ref[...] = jax.nn.sigmoid(logits)


def kernel(x, edge_index, W1, b1, Wmu, bmu, Wlv, blv):
    src = edge_index[0]
    dst = edge_index[1]

    dst_d = dst.reshape(NC * NS, NCHD, KD)
    src2 = jnp.stack([(src * 2).reshape(NS, NCHK, K),
                      (src * 2 + 1).reshape(NS, NCHK, K)])
    dst_s = dst.reshape(NS, NCHK, K)
    ones16 = jnp.ones((KD, 16), jnp.float32)
    zeros16 = jnp.zeros((RPT, 16), jnp.float32)
    zerosF = jnp.zeros((RPT, FH), jnp.float32)
    Wcat = jnp.concatenate([Wmu, Wlv], axis=1)
    b1r = b1.reshape(1, F)
    bcat = jnp.concatenate([bmu, blv]).reshape(1, F)

    degp = _sc_degree(dst_d, ones16, zeros16).reshape(NC, N, 16)

    y1 = pl.pallas_call(
        _tc_xw_body,
        grid=(N // _BR,),
        in_specs=[
            pl.BlockSpec((_BR, F), lambda i: (i, 0)),
            pl.BlockSpec((F, F), lambda i: (0, 0)),
            pl.BlockSpec((2, _BR, 16), lambda i: (0, i, 0)),
        ],
        out_specs=pl.BlockSpec((_BR, F), lambda i: (i, 0)),
        out_shape=jax.ShapeDtypeStruct((N, F), jnp.float32),
    )(x, W1, degp)

    s1 = _sc_scatter(y1.reshape(2 * N, FH), src2, dst_s, zerosF)
    s1 = s1.reshape(NC, N, FH)

    y2 = pl.pallas_call(
        _tc_layer2_body,
        grid=(N // _BR,),
        in_specs=[
            pl.BlockSpec((2, _BR, FH), lambda i: (0, i, 0)),
            pl.BlockSpec((_BR, F), lambda i: (i, 0)),
            pl.BlockSpec((2, _BR, 16), lambda i: (0, i, 0)),
            pl.BlockSpec((1, F), lambda i: (0, 0)),
            pl.BlockSpec((F, F), lambda i: (0, 0)),
        ],
        out_specs=pl.BlockSpec((_BR, F), lambda i: (i, 0)),
        out_shape=jax.ShapeDtypeStruct((N, F), jnp.float32),
    )(s1, y1, degp, b1r, Wcat)

    s2 = _sc_scatter(y2.reshape(2 * N, FH), src2, dst_s, zerosF)
    s2 = s2.reshape(NC, N, FH)

    mulv = pl.pallas_call(
        _tc_out_body,
        grid=(N // _BR,),
        in_specs=[
            pl.BlockSpec((2, _BR, FH), lambda i: (0, i, 0)),
            pl.BlockSpec((_BR, F), lambda i: (i, 0)),
            pl.BlockSpec((2, _BR, 16), lambda i: (0, i, 0)),
            pl.BlockSpec((1, F), lambda i: (0, 0)),
        ],
        out_specs=pl.BlockSpec((_BR, F), lambda i: (i, 0)),
        out_shape=jax.ShapeDtypeStruct((N, F), jnp.float32),
    )(s2, y2, degp, bcat)

    mu = mulv[:, :H2]
    logvar = mulv[:, H2:]

    adj = pl.pallas_call(
        _tc_dec_body,
        grid=(N // _BD_I, pl.cdiv(N, _BD_J)),
        in_specs=[
            pl.BlockSpec((_BD_I, H2), lambda i, j: (i, 0)),
            pl.BlockSpec((_BD_J, H2), lambda i, j: (j, 0)),
        ],
        out_specs=pl.BlockSpec((_BD_I, _BD_J), lambda i, j: (i, j)),
        out_shape=jax.ShapeDtypeStruct((N, N), jnp.float32),
    )(mu, mu)

    return (adj, mu, logvar)


# speedup vs baseline: 1.1451x; 1.1451x over previous
import functools

import jax
import jax.numpy as jnp
from jax import lax
from jax.experimental import pallas as pl
from jax.experimental.pallas import tpu as pltpu
from jax.experimental.pallas import tpu_sc as plsc

N = 10000
E = 160000
F = 256
H2 = 128
FH = 128
NC, NS = 2, 16
RPT = N // NS

EPW = E // (NC * NS)
KD = 40
NCHD = EPW // KD

EPT = E // NS
K = 80
NCHK = EPT // K

_MESH = plsc.VectorSubcoreMesh(core_axis_name="c", subcore_axis_name="s",
                               num_cores=NC, num_subcores=NS)


@functools.partial(
    pl.kernel,
    out_type=jax.ShapeDtypeStruct((NC * NS, RPT, 16), jnp.float32),
    mesh=_MESH,
    scratch_types=[
        pltpu.VMEM((NCHD, KD), jnp.int32),
        pltpu.VMEM((KD, 16), jnp.float32),
        pltpu.VMEM_SHARED((N, 16), jnp.float32),
    ],
    compiler_params=pltpu.CompilerParams(use_tc_tiling_on_sc=False),
)
def _sc_degree(dst_hbm, ones_hbm, zeros_hbm, out_hbm, dst_v, ones_v, acc):
    c = lax.axis_index("c")
    s = lax.axis_index("s")
    wid = c * NS + s
    pltpu.sync_copy(dst_hbm.at[wid], dst_v)
    pltpu.sync_copy(ones_hbm, ones_v)
    pltpu.sync_copy(zeros_hbm, acc.at[pl.ds(s * RPT, RPT)])
    plsc.subcore_barrier()

    def body(j, carry):
        pltpu.sync_copy(ones_v, acc.at[dst_v.at[j]], add=True)
        return carry

    lax.fori_loop(0, NCHD, body, 0)
    plsc.subcore_barrier()
    pltpu.sync_copy(acc.at[pl.ds(s * RPT, RPT)], out_hbm.at[wid])


@functools.partial(
    pl.kernel,
    out_type=jax.ShapeDtypeStruct((NC * NS, RPT, FH), jnp.float32),
    mesh=_MESH,
    scratch_types=[
        pltpu.VMEM((NCHK, K), jnp.int32),
        pltpu.VMEM((NCHK, K), jnp.int32),
        pltpu.VMEM((K, FH), jnp.float32),
        pltpu.VMEM((K, FH), jnp.float32),
        pltpu.VMEM_SHARED((N, FH), jnp.float32),
        pltpu.SemaphoreType.DMA,
        pltpu.SemaphoreType.DMA,
    ],
    compiler_params=pltpu.CompilerParams(use_tc_tiling_on_sc=False),
)
def _sc_scatter(y_hbm, src2_hbm, dst_hbm, zeros_hbm, out_hbm,
                src_v, dst_v, rows0, rows1, acc, sem0, sem1):
    c = lax.axis_index("c")
    s = lax.axis_index("s")
    wid = c * NS + s
    pltpu.sync_copy(src2_hbm.at[c, s], src_v)
    pltpu.sync_copy(dst_hbm.at[s], dst_v)
    pltpu.sync_copy(zeros_hbm, acc.at[pl.ds(s * RPT, RPT)])
    plsc.subcore_barrier()

    def gather(rows, sem, j):
        pltpu.async_copy(y_hbm.at[src_v.at[j]], rows, sem)

    def wait_scatter(rows, sem, j):
        pltpu.make_async_copy(y_hbm.at[src_v.at[j]], rows, sem).wait()
        pltpu.sync_copy(rows, acc.at[dst_v.at[j]], add=True)

    gather(rows0, sem0, 0)

    def body(t2, carry):
        t = 2 * t2
        gather(rows1, sem1, t + 1)
        wait_scatter(rows0, sem0, t)
        gather(rows0, sem0, t + 2)
        wait_scatter(rows1, sem1, t + 1)
        return carry

    lax.fori_loop(0, (NCHK - 1) // 2, body, 0)
    wait_scatter(rows0, sem0, NCHK - 1)
    plsc.subcore_barrier()
    pltpu.sync_copy(acc.at[pl.ds(s * RPT, RPT)], out_hbm.at[wid])


_BR = 1000


def _dinv(dp):
    return lax.rsqrt(dp[0, :, :1] + dp[1, :, :1] + 1.0)


def _tc_xw_body(x_ref, w_ref, dp_ref, y_ref):
    dinv = _dinv(dp_ref[...])
    y_ref[...] = jnp.dot(x_ref[...], w_ref[...],
                         preferred_element_type=jnp.float32) * dinv


def _tc_layer2_body(s1_ref, y1_ref, dp_ref, b1_ref, w_ref, y2_ref):
    dinv = _dinv(dp_ref[...])
    s1 = jnp.concatenate([s1_ref[0], s1_ref[1]], axis=-1)
    h = jnp.maximum(dinv * (s1 + y1_ref[...]) + b1_ref[...], 0.0)
    y2_ref[...] = jnp.dot(h, w_ref[...],
                          preferred_element_type=jnp.float32) * dinv


def _tc_out_body(s2_ref, y2_ref, dp_ref, b_ref, o_ref):
    dinv = _dinv(dp_ref[...])
    s2 = jnp.concatenate([s2_ref[0], s2_ref[1]], axis=-1)
    o_ref[...] = dinv * (s2 + y2_ref[...]) + b_ref[...]


_BD_I = 2000
_BD_J = 2560


def _tc_dec_body(a_ref, b_ref, o_ref):
    logits = lax.dot_general(a_ref[...], b_ref[...],
                             (((1,), (1,)), ((), ())),
                             preferred_element_type=jnp.float32)
    o_ref[...] = jax.nn.sigmoid(logits)


def kernel(x, edge_index, W1, b1, Wmu, bmu, Wlv, blv):
    src = edge_index[0]
    dst = edge_index[1]

    dst_d = dst.reshape(NC * NS, NCHD, KD)
    src2 = jnp.stack([(src * 2).reshape(NS, NCHK, K),
                      (src * 2 + 1).reshape(NS, NCHK, K)])
    dst_s = dst.reshape(NS, NCHK, K)
    ones16 = jnp.ones((KD, 16), jnp.float32)
    zeros16 = jnp.zeros((RPT, 16), jnp.float32)
    zerosF = jnp.zeros((RPT, FH), jnp.float32)
    Wcat = jnp.concatenate([Wmu, Wlv], axis=1)
    b1r = b1.reshape(1, F)
    bcat = jnp.concatenate([bmu, blv]).reshape(1, F)

    degp = _sc_degree(dst_d, ones16, zeros16).reshape(NC, N, 16)

    y1 = pl.pallas_call(
        _tc_xw_body,
        grid=(N // _BR,),
        in_specs=[
            pl.BlockSpec((_BR, F), lambda i: (i, 0)),
            pl.BlockSpec((F, F), lambda i: (0, 0)),
            pl.BlockSpec((2, _BR, 16), lambda i: (0, i, 0)),
        ],
        out_specs=pl.BlockSpec((_BR, F), lambda i: (i, 0)),
        out_shape=jax.ShapeDtypeStruct((N, F), jnp.float32),
    )(x, W1, degp)

    s1 = _sc_scatter(y1.reshape(2 * N, FH), src2, dst_s, zerosF)
    s1 = s1.reshape(NC, N, FH)

    y2 = pl.pallas_call(
        _tc_layer2_body,
        grid=(N // _BR,),
        in_specs=[
            pl.BlockSpec((2, _BR, FH), lambda i: (0, i, 0)),
            pl.BlockSpec((_BR, F), lambda i: (i, 0)),
            pl.BlockSpec((2, _BR, 16), lambda i: (0, i, 0)),
            pl.BlockSpec((1, F), lambda i: (0, 0)),
            pl.BlockSpec((F, F), lambda i: (0, 0)),
        ],
        out_specs=pl.BlockSpec((_BR, F), lambda i: (i, 0)),
        out_shape=jax.ShapeDtypeStruct((N, F), jnp.float32),
    )(s1, y1, degp, b1r, Wcat)

    s2 = _sc_scatter(y2.reshape(2 * N, FH), src2, dst_s, zerosF)
    s2 = s2.reshape(NC, N, FH)

    mulv = pl.pallas_call(
        _tc_out_body,
        grid=(N // _BR,),
        in_specs=[
            pl.BlockSpec((2, _BR, FH), lambda i: (0, i, 0)),
            pl.BlockSpec((_BR, F), lambda i: (i, 0)),
            pl.BlockSpec((2, _BR, 16), lambda i: (0, i, 0)),
            pl.BlockSpec((1, F), lambda i: (0, 0)),
        ],
        out_specs=pl.BlockSpec((_BR, F), lambda i: (i, 0)),
        out_shape=jax.ShapeDtypeStruct((N, F), jnp.float32),
    )(s2, y2, degp, bcat)

    mu = mulv[:, :H2]
    logvar = mulv[:, H2:]

    adj = pl.pallas_call(
        _tc_dec_body,
        grid=(N // _BD_I, pl.cdiv(N, _BD_J)),
        in_specs=[
            pl.BlockSpec((_BD_I, H2), lambda i, j: (i, 0)),
            pl.BlockSpec((_BD_J, H2), lambda i, j: (j, 0)),
        ],
        out_specs=pl.BlockSpec((_BD_I, _BD_J), lambda i, j: (i, j)),
        out_shape=jax.ShapeDtypeStruct((N, N), jnp.float32),
    )(mu, mu)

    return (adj, mu, logvar)


# speedup vs baseline: 1.2060x; 1.0532x over previous
import functools

import jax
import jax.numpy as jnp
from jax import lax
from jax.experimental import pallas as pl
from jax.experimental.pallas import tpu as pltpu
from jax.experimental.pallas import tpu_sc as plsc

N = 10000
E = 160000
F = 256
H2 = 128
FH = 128
NC, NS = 2, 16
RPT = N // NS

EPW = E // (NC * NS)
KD = 40
NCHD = EPW // KD

EPT = E // NS
K = 80
NCHK = EPT // K

_MESH = plsc.VectorSubcoreMesh(core_axis_name="c", subcore_axis_name="s",
                               num_cores=NC, num_subcores=NS)


@functools.partial(
    pl.kernel,
    out_type=jax.ShapeDtypeStruct((NC * NS, RPT, 16), jnp.float32),
    mesh=_MESH,
    scratch_types=[
        pltpu.VMEM((NCHD, KD), jnp.int32),
        pltpu.VMEM((KD, 16), jnp.float32),
        pltpu.VMEM_SHARED((N, 16), jnp.float32),
    ],
    compiler_params=pltpu.CompilerParams(use_tc_tiling_on_sc=False),
)
def _sc_degree(dst_hbm, ones_hbm, zeros_hbm, out_hbm, dst_v, ones_v, acc):
    c = lax.axis_index("c")
    s = lax.axis_index("s")
    wid = c * NS + s
    pltpu.sync_copy(dst_hbm.at[wid], dst_v)
    pltpu.sync_copy(ones_hbm, ones_v)
    pltpu.sync_copy(zeros_hbm, acc.at[pl.ds(s * RPT, RPT)])
    plsc.subcore_barrier()

    def body(j, carry):
        pltpu.sync_copy(ones_v, acc.at[dst_v.at[j]], add=True)
        return carry

    lax.fori_loop(0, NCHD, body, 0)
    plsc.subcore_barrier()
    pltpu.sync_copy(acc.at[pl.ds(s * RPT, RPT)], out_hbm.at[wid])


@functools.partial(
    pl.kernel,
    out_type=jax.ShapeDtypeStruct((NC * NS, RPT, FH), jnp.float32),
    mesh=_MESH,
    scratch_types=[
        pltpu.VMEM((NCHK, K), jnp.int32),
        pltpu.VMEM((NCHK, K), jnp.int32),
        pltpu.VMEM((K, FH), jnp.float32),
        pltpu.VMEM((K, FH), jnp.float32),
        pltpu.VMEM_SHARED((N, FH), jnp.float32),
        pltpu.SemaphoreType.DMA,
        pltpu.SemaphoreType.DMA,
    ],
    compiler_params=pltpu.CompilerParams(use_tc_tiling_on_sc=False),
)
def _sc_scatter(y_hbm, src_hbm, dst_hbm, zeros_hbm, out_hbm,
                src_v, dst_v, rows0, rows1, acc, sem0, sem1):
    c = lax.axis_index("c")
    s = lax.axis_index("s")
    wid = c * NS + s
    pltpu.sync_copy(src_hbm.at[c, s], src_v)
    pltpu.sync_copy(dst_hbm.at[s], dst_v)
    pltpu.sync_copy(zeros_hbm, acc.at[pl.ds(s * RPT, RPT)])
    plsc.subcore_barrier()

    def gather(rows, sem, j):
        pltpu.async_copy(y_hbm.at[src_v.at[j]], rows, sem)

    def wait_scatter(rows, sem, j):
        pltpu.make_async_copy(y_hbm.at[src_v.at[j]], rows, sem).wait()
        pltpu.sync_copy(rows, acc.at[dst_v.at[j]], add=True)

    gather(rows0, sem0, 0)

    def body(t2, carry):
        t = 2 * t2
        gather(rows1, sem1, t + 1)
        wait_scatter(rows0, sem0, t)
        gather(rows0, sem0, t + 2)
        wait_scatter(rows1, sem1, t + 1)
        return carry

    lax.fori_loop(0, (NCHK - 1) // 2, body, 0)
    wait_scatter(rows0, sem0, NCHK - 1)
    plsc.subcore_barrier()
    pltpu.sync_copy(acc.at[pl.ds(s * RPT, RPT)], out_hbm.at[wid])


_BR = 1000


def _dinv(dp):
    return lax.rsqrt(dp[0, :, :1] + dp[1, :, :1] + 1.0)


def _tc_xw_body(x_ref, w_ref, dp_ref, y_ref):
    dinv = _dinv(dp_ref[...])
    y = jnp.dot(x_ref[...], w_ref[...],
                preferred_element_type=jnp.float32) * dinv
    y_ref[0] = y[:, :FH]
    y_ref[1] = y[:, FH:]


def _tc_layer2_body(s1_ref, y1_ref, dp_ref, b1_ref, w_ref, y2_ref):
    dinv = _dinv(dp_ref[...])
    s1 = jnp.concatenate([s1_ref[0], s1_ref[1]], axis=-1)
    y1 = jnp.concatenate([y1_ref[0], y1_ref[1]], axis=-1)
    h = jnp.maximum(dinv * (s1 + y1) + b1_ref[...], 0.0)
    y2 = jnp.dot(h, w_ref[...], preferred_element_type=jnp.float32) * dinv
    y2_ref[0] = y2[:, :FH]
    y2_ref[1] = y2[:, FH:]


def _tc_out_body(s2_ref, y2_ref, dp_ref, b_ref, o_ref):
    dinv = _dinv(dp_ref[...])
    s2 = jnp.concatenate([s2_ref[0], s2_ref[1]], axis=-1)
    y2 = jnp.concatenate([y2_ref[0], y2_ref[1]], axis=-1)
    o_ref[...] = dinv * (s2 + y2) + b_ref[...]


_BD_I = 2000
_BD_J = 2048


def _tc_dec_body(a_ref, b_ref, o_ref):
    logits = lax.dot_general(a_ref[...], b_ref[...],
                             (((1,), (1,)), ((), ())),
                             preferred_element_type=jnp.float32)
    o_ref[...] = jax.nn.sigmoid(logits)


def kernel(x, edge_index, W1, b1, Wmu, bmu, Wlv, blv):
    src = edge_index[0]
    dst = edge_index[1]

    dst_d = dst.reshape(NC * NS, NCHD, KD)
    src_s = jnp.stack([src.reshape(NS, NCHK, K),
                       (src + N).reshape(NS, NCHK, K)])
    dst_s = dst.reshape(NS, NCHK, K)
    ones16 = jnp.ones((KD, 16), jnp.float32)
    zeros16 = jnp.zeros((RPT, 16), jnp.float32)
    zerosF = jnp.zeros((RPT, FH), jnp.float32)
    Wcat = jnp.concatenate([Wmu, Wlv], axis=1)
    b1r = b1.reshape(1, F)
    bcat = jnp.concatenate([bmu, blv]).reshape(1, F)

    degp = _sc_degree(dst_d, ones16, zeros16).reshape(NC, N, 16)

    y1 = pl.pallas_call(
        _tc_xw_body,
        grid=(N // _BR,),
        in_specs=[
            pl.BlockSpec((_BR, F), lambda i: (i, 0)),
            pl.BlockSpec((F, F), lambda i: (0, 0)),
            pl.BlockSpec((2, _BR, 16), lambda i: (0, i, 0)),
        ],
        out_specs=pl.BlockSpec((2, _BR, FH), lambda i: (0, i, 0)),
        out_shape=jax.ShapeDtypeStruct((2, N, FH), jnp.float32),
    )(x, W1, degp)

    s1 = _sc_scatter(y1.reshape(2 * N, FH), src_s, dst_s, zerosF)
    s1 = s1.reshape(NC, N, FH)

    y2 = pl.pallas_call(
        _tc_layer2_body,
        grid=(N // _BR,),
        in_specs=[
            pl.BlockSpec((2, _BR, FH), lambda i: (0, i, 0)),
            pl.BlockSpec((2, _BR, FH), lambda i: (0, i, 0)),
            pl.BlockSpec((2, _BR, 16), lambda i: (0, i, 0)),
            pl.BlockSpec((1, F), lambda i: (0, 0)),
            pl.BlockSpec((F, F), lambda i: (0, 0)),
        ],
        out_specs=pl.BlockSpec((2, _BR, FH), lambda i: (0, i, 0)),
        out_shape=jax.ShapeDtypeStruct((2, N, FH), jnp.float32),
    )(s1, y1, degp, b1r, Wcat)

    s2 = _sc_scatter(y2.reshape(2 * N, FH), src_s, dst_s, zerosF)
    s2 = s2.reshape(NC, N, FH)

    mulv = pl.pallas_call(
        _tc_out_body,
        grid=(N // _BR,),
        in_specs=[
            pl.BlockSpec((2, _BR, FH), lambda i: (0, i, 0)),
            pl.BlockSpec((2, _BR, FH), lambda i: (0, i, 0)),
            pl.BlockSpec((2, _BR, 16), lambda i: (0, i, 0)),
            pl.BlockSpec((1, F), lambda i: (0, 0)),
        ],
        out_specs=pl.BlockSpec((_BR, F), lambda i: (i, 0)),
        out_shape=jax.ShapeDtypeStruct((N, F), jnp.float32),
    )(s2, y2, degp, bcat)

    mu = mulv[:, :H2]
    logvar = mulv[:, H2:]

    adj = pl.pallas_call(
        _tc_dec_body,
        grid=(N // _BD_I, pl.cdiv(N, _BD_J)),
        in_specs=[
            pl.BlockSpec((_BD_I, H2), lambda i, j: (i, 0)),
            pl.BlockSpec((_BD_J, H2), lambda i, j: (j, 0)),
        ],
        out_specs=pl.BlockSpec((_BD_I, _BD_J), lambda i, j: (i, j)),
        out_shape=jax.ShapeDtypeStruct((N, N), jnp.float32),
    )(mu, mu)

    return (adj, mu, logvar)
